# Initial kernel scaffold; baseline (speedup 1.0000x reference)
import jax
import jax.numpy as jnp
from jax.experimental import pallas as pl


def kernel(x, edge_index, edge_type, W1, root1, b1, W2, root2, b2, W3, root3, b3):
    raise NotImplementedError("write your pallas kernel here")


# speedup vs baseline: 9.9621x; 9.9621x over previous
import functools

import jax
import jax.numpy as jnp
from jax import lax
from jax.experimental import pallas as pl
from jax.experimental.pallas import tpu as pltpu
from jax.experimental.pallas import tpu_sc as plsc

N = 50000
E = 800000
R = 6
NC = 2
NS = 16
NW = NC * NS
NPADR = 50176
TPW = NPADR // NW
EPW = NPADR
EPAD = EPW * NS
ECH = 1568
NCH = EPW // ECH
NCHG = EPAD // ECH
VSTEPS = ECH // 16
BATCH = 64
SCAP = 1792
STRASH = SCAP - 16
DUMMY_DST = 50000
ONES_COL = 32
RT = R * NPADR + NW * 8

_MESH = dict(core_axis_name="c", subcore_axis_name="s", num_cores=NC,
             num_subcores=NS)


def _zero16():
    return jnp.zeros((16,), jnp.float32)


def _compact16(stage_s, stage_t, s16, loc, m, f):
    pos = plsc.cumsum(m.astype(jnp.int32)) - 1
    lane = lax.iota(jnp.int32, 16)
    idx = jnp.where(m, f + pos, STRASH + lane)
    if stage_s is not None:
        plsc.store_scatter(stage_s, [idx], s16)
    plsc.store_scatter(stage_t, [idx], loc)
    return f + plsc.all_reduce_population_count(m)


@functools.lru_cache(maxsize=None)
def _make_prep():
    mesh = plsc.VectorSubcoreMesh(**_MESH)

    @functools.partial(
        pl.kernel,
        out_type=jax.ShapeDtypeStruct((EPAD,), jnp.int32),
        mesh=mesh,
        compiler_params=pltpu.CompilerParams(needs_layout_passes=False),
        scratch_types=[
            pltpu.VMEM((ECH,), jnp.int32),
            pltpu.VMEM((ECH,), jnp.int32),
            pltpu.VMEM((ECH,), jnp.int32),
        ],
    )
    def prep(edst, et, slots_out, dstb, typb, slotb):
        sc = lax.axis_index("c")
        sid = lax.axis_index("s")
        ebase = sid * EPW

        def chunk(ci, _):
            c = sc * (NCH // 2) + ci
            off = ebase + c * ECH
            pltpu.sync_copy(edst.at[pl.ds(off, ECH)], dstb)
            pltpu.sync_copy(et.at[pl.ds(off, ECH)], typb)

            def v(i, _):
                d16 = dstb[pl.ds(i * 16, 16)]
                t16 = typb[pl.ds(i * 16, 16)]
                slotb[pl.ds(i * 16, 16)] = (t16 << 16) | d16
                return 0
            lax.fori_loop(0, VSTEPS, v, 0)
            pltpu.sync_copy(slotb, slots_out.at[pl.ds(off, ECH)])
            return 0
        lax.fori_loop(0, NCH // 2, chunk, 0)

    return prep


@functools.lru_cache(maxsize=None)
def _make_agg(din):
    zr = 56
    nz = TPW // zr
    mesh = plsc.VectorSubcoreMesh(**_MESH)

    @functools.partial(
        pl.kernel,
        out_type=jax.ShapeDtypeStruct((RT, din), jnp.float32),
        mesh=mesh,
        compiler_params=pltpu.CompilerParams(needs_layout_passes=False),
        scratch_types=[
            pltpu.VMEM((ECH,), jnp.int32),
            pltpu.VMEM((ECH,), jnp.int32),
            pltpu.VMEM((SCAP,), jnp.int32),
            pltpu.VMEM((SCAP,), jnp.int32),
            pltpu.VMEM((BATCH,), jnp.int32),
            pltpu.VMEM((BATCH,), jnp.int32),
            pltpu.VMEM((BATCH, din), jnp.float32),
            pltpu.VMEM((zr, din), jnp.float32),
            pltpu.SemaphoreType.DMA,
        ],
    )
    def agg(xh, esrc, slots, acc_out,
            srcb, slotb, stage_s, stage_t, gsrc, gslot, rows, zbuf, gsem):
        sc = lax.axis_index("c")
        sid = lax.axis_index("s")
        w = sid * NC + sc
        dlo = w * TPW

        def fill_z(j, _):
            for q in range(din // 16):
                zbuf[j, pl.ds(q * 16, 16)] = _zero16()
            return 0
        lax.fori_loop(0, zr, fill_z, 0)

        for r in range(R):
            base = r * NPADR + dlo
            for zi in range(nz):
                pltpu.sync_copy(zbuf, acc_out.at[pl.ds(base + zi * zr, zr)])
        trash = R * NPADR + w * 8
        pltpu.sync_copy(zbuf.at[pl.ds(0, 8)], acc_out.at[pl.ds(trash, 8)])

        def fire(nb):
            def batch(b, _):
                for j in range(BATCH // 16):
                    gsrc[pl.ds(16 * j, 16)] = (
                        stage_s[pl.ds(BATCH * b + 16 * j, 16)])
                    gslot[pl.ds(16 * j, 16)] = (
                        stage_t[pl.ds(BATCH * b + 16 * j, 16)])
                pltpu.async_copy(xh.at[gsrc], rows, gsem).wait()
                pltpu.sync_copy(rows, acc_out.at[gslot], add=True)
                return 0
            lax.fori_loop(0, nb, batch, 0)

        def chunk(c, f):
            off = c * ECH
            pltpu.sync_copy(esrc.at[pl.ds(off, ECH)], srcb)
            pltpu.sync_copy(slots.at[pl.ds(off, ECH)], slotb)

            def v(i, f):
                s16 = srcb[pl.ds(i * 16, 16)]
                t16 = slotb[pl.ds(i * 16, 16)]
                d16 = t16 & 0xFFFF
                r16 = lax.shift_right_logical(t16, 16)
                m = (d16 >= dlo) & (d16 < dlo + TPW)
                loc = r16 * NPADR + d16
                return _compact16(stage_s, stage_t, s16, loc, m, f)
            fv = lax.fori_loop(0, VSTEPS, v, f)
            fs = fv[0]
            nb = fs // BATCH
            fire(nb)
            rem = fs - nb * BATCH
            for j in range(BATCH // 16):
                @pl.when(16 * j < rem)
                def _():
                    stage_s[pl.ds(16 * j, 16)] = (
                        stage_s[pl.ds(nb * BATCH + 16 * j, 16)])
                    stage_t[pl.ds(16 * j, 16)] = (
                        stage_t[pl.ds(nb * BATCH + 16 * j, 16)])
            return jnp.full((16,), rem, jnp.int32)
        fv = lax.fori_loop(0, NCHG, chunk, jnp.zeros((16,), jnp.int32))
        f = fv[0]

        nb = (f + BATCH - 1) // BATCH
        pe = nb * BATCH
        dums = jnp.zeros((16,), jnp.int32)
        dumt = jnp.full((16,), trash, jnp.int32)
        for j in range(BATCH // 16):
            @pl.when(f + 16 * j < pe)
            def _():
                stage_s[pl.ds(f + 16 * j, 16)] = dums
                stage_t[pl.ds(f + 16 * j, 16)] = dumt
        fire(nb)

    return agg


NB = 400
NBLK = N // NB


def _layer_body(x_ref, acc_ref, cnt_ref, w_ref, b_ref):
    inv = 1.0 / jnp.maximum(cnt_ref[...], 1.0)
    o = jnp.dot(x_ref[...], w_ref[0], preferred_element_type=jnp.float32)
    for r in range(R):
        o = o + jnp.dot(acc_ref[r] * inv[r][:, ONES_COL:ONES_COL + 1],
                        w_ref[r + 1], preferred_element_type=jnp.float32)
    return jnp.maximum(o + b_ref[...], 0.0)


@functools.lru_cache(maxsize=None)
def _make_layer(din, dout, final):
    def body(x_ref, acc_ref, cnt_ref, w_ref, b_ref, o_ref):
        h = _layer_body(x_ref, acc_ref, cnt_ref, w_ref, b_ref)
        if not final:
            o_ref[...] = h
        else:
            i = pl.program_id(0)
            t = jnp.max(jnp.tanh(h), axis=0, keepdims=True)

            @pl.when(i == 0)
            def _():
                o_ref[...] = jnp.full((1, dout), -2.0, jnp.float32)
            o_ref[...] = jnp.maximum(o_ref[...], t)

            @pl.when(i == NBLK - 1)
            def _():
                z = o_ref[...]
                o_ref[...] = z * lax.rsqrt(jnp.sum(z * z))

    out_shape = jax.ShapeDtypeStruct((1, dout) if final else (N, dout),
                                     jnp.float32)
    out_spec = (pl.BlockSpec((1, dout), lambda i: (0, 0)) if final
                else pl.BlockSpec((NB, dout), lambda i: (i, 0)))
    return pl.pallas_call(
        body,
        grid=(NBLK,),
        in_specs=[
            pl.BlockSpec((NB, din), lambda i: (i, 0)),
            pl.BlockSpec((R, NB, din), lambda i: (0, i, 0)),
            pl.BlockSpec((R, NB, 128), lambda i: (0, i, 0)),
            pl.BlockSpec((R + 1, din, dout), lambda i: (0, 0, 0)),
            pl.BlockSpec((1, dout), lambda i: (0, 0)),
        ],
        out_specs=out_spec,
        out_shape=out_shape,
        compiler_params=pltpu.CompilerParams(
            dimension_semantics=("arbitrary",)),
    )


def _stack_w(root, w, din):
    pad = din - root.shape[0]
    rootp = jnp.pad(root, ((0, pad), (0, 0)))
    wp = jnp.pad(w, ((0, 0), (0, pad), (0, 0)))
    return jnp.concatenate([rootp[None], wp], axis=0)


def kernel(x, edge_index, edge_type, W1, root1, b1, W2, root2, b2,
           W3, root3, b3):
    x256 = jnp.pad(x, ((0, 0), (0, 256 - x.shape[1])))
    x256 = x256.at[:, ONES_COL].set(1.0)
    npadE = EPAD - E
    esrc = jnp.concatenate([edge_index[0], jnp.zeros((npadE,), jnp.int32)])
    edst = jnp.concatenate(
        [edge_index[1], jnp.full((npadE,), DUMMY_DST, jnp.int32)])
    et = jnp.concatenate([edge_type, jnp.zeros((npadE,), jnp.int32)])

    slots = _make_prep()(edst, et)

    acc1 = _make_agg(256)(x256, esrc, slots)[:R * NPADR].reshape(
        R, NPADR, 256)
    cnt = acc1
    h1 = _make_layer(256, 256, False)(
        x256, acc1, cnt, _stack_w(root1, W1, 256), b1[None])

    acc2 = _make_agg(256)(h1, esrc, slots)[:R * NPADR].reshape(
        R, NPADR, 256)
    h2 = _make_layer(256, 256, False)(
        h1, acc2, cnt, _stack_w(root2, W2, 256), b2[None])

    acc3 = _make_agg(256)(h2, esrc, slots)[:R * NPADR].reshape(
        R, NPADR, 256)
    z = _make_layer(256, 512, True)(
        h2, acc3, cnt, _stack_w(root3, W3, 256), b3[None])
    return z


# speedup vs baseline: 10.8546x; 1.0896x over previous
import functools

import jax
import jax.numpy as jnp
from jax import lax
from jax.experimental import pallas as pl
from jax.experimental.pallas import tpu as pltpu
from jax.experimental.pallas import tpu_sc as plsc

N = 50000
E = 800000
R = 6
NC = 2
NS = 16
NW = NC * NS
NPADR = 50176
TPW = NPADR // NW
EPW = NPADR
EPAD = EPW * NS
ECH = 1568
NCH = EPW // ECH
NCHG = EPAD // ECH
VSTEPS = ECH // 16
BATCH = 128
GRP = 16
NGRP = NCHG // GRP
SCAP = GRP * ECH + 256
STRASH = SCAP - 16
DUMMY_DST = 50000
ONES_COL = 32
RT = R * NPADR + NW * 8

_MESH = dict(core_axis_name="c", subcore_axis_name="s", num_cores=NC,
             num_subcores=NS)


def _zero16():
    return jnp.zeros((16,), jnp.float32)


def _compact16(stage_s, stage_t, s16, loc, m, f):
    pos = plsc.cumsum(m.astype(jnp.int32)) - 1
    lane = lax.iota(jnp.int32, 16)
    idx = jnp.where(m, f + pos, STRASH + lane)
    if stage_s is not None:
        plsc.store_scatter(stage_s, [idx], s16)
    plsc.store_scatter(stage_t, [idx], loc)
    return f + plsc.all_reduce_population_count(m)


@functools.lru_cache(maxsize=None)
def _make_prep():
    mesh = plsc.VectorSubcoreMesh(**_MESH)

    @functools.partial(
        pl.kernel,
        out_type=jax.ShapeDtypeStruct((EPAD,), jnp.int32),
        mesh=mesh,
        compiler_params=pltpu.CompilerParams(needs_layout_passes=False),
        scratch_types=[
            pltpu.VMEM((ECH,), jnp.int32),
            pltpu.VMEM((ECH,), jnp.int32),
            pltpu.VMEM((ECH,), jnp.int32),
        ],
    )
    def prep(edst, et, slots_out, dstb, typb, slotb):
        sc = lax.axis_index("c")
        sid = lax.axis_index("s")
        ebase = sid * EPW

        def chunk(ci, _):
            c = sc * (NCH // 2) + ci
            off = ebase + c * ECH
            pltpu.sync_copy(edst.at[pl.ds(off, ECH)], dstb)
            pltpu.sync_copy(et.at[pl.ds(off, ECH)], typb)

            def v(i, _):
                d16 = dstb[pl.ds(i * 16, 16)]
                t16 = typb[pl.ds(i * 16, 16)]
                slotb[pl.ds(i * 16, 16)] = (t16 << 16) | d16
                return 0
            lax.fori_loop(0, VSTEPS, v, 0)
            pltpu.sync_copy(slotb, slots_out.at[pl.ds(off, ECH)])
            return 0
        lax.fori_loop(0, NCH // 2, chunk, 0)

    return prep


@functools.lru_cache(maxsize=None)
def _make_agg(din):
    zr = 16
    nz = TPW // zr
    mesh = plsc.VectorSubcoreMesh(**_MESH)

    @functools.partial(
        pl.kernel,
        out_type=jax.ShapeDtypeStruct((RT, din), jnp.float32),
        mesh=mesh,
        compiler_params=pltpu.CompilerParams(needs_layout_passes=False),
        scratch_types=[
            pltpu.VMEM((ECH,), jnp.int32),
            pltpu.VMEM((ECH,), jnp.int32),
            pltpu.VMEM((SCAP,), jnp.int32),
            pltpu.VMEM((SCAP,), jnp.int32),
            pltpu.VMEM((BATCH,), jnp.int32),
            pltpu.VMEM((BATCH,), jnp.int32),
            pltpu.VMEM((BATCH,), jnp.int32),
            pltpu.VMEM((BATCH,), jnp.int32),
            pltpu.VMEM((BATCH, din), jnp.float32),
            pltpu.VMEM((BATCH, din), jnp.float32),
            pltpu.VMEM((zr, din), jnp.float32),
            pltpu.SemaphoreType.DMA,
            pltpu.SemaphoreType.DMA,
        ],
    )
    def agg(xh, esrc, slots, acc_out,
            srcb, slotb, stage_s, stage_t, gsrc0, gslot0, gsrc1, gslot1,
            rows0, rows1, zbuf, sem0, sem1):
        sc = lax.axis_index("c")
        sid = lax.axis_index("s")
        w = sid * NC + sc
        dlo = w * TPW

        def fill_z(j, _):
            for q in range(din // 16):
                zbuf[j, pl.ds(q * 16, 16)] = _zero16()
            return 0
        lax.fori_loop(0, zr, fill_z, 0)

        for r in range(R):
            base = r * NPADR + dlo
            for zi in range(nz):
                pltpu.sync_copy(zbuf, acc_out.at[pl.ds(base + zi * zr, zr)])
        trash = R * NPADR + w * 8
        pltpu.sync_copy(zbuf.at[pl.ds(0, 8)], acc_out.at[pl.ds(trash, 8)])

        def issue(b, g, t, rws, sem):
            for j in range(BATCH // 16):
                g[pl.ds(16 * j, 16)] = stage_s[pl.ds(BATCH * b + 16 * j, 16)]
                t[pl.ds(16 * j, 16)] = stage_t[pl.ds(BATCH * b + 16 * j, 16)]
            pltpu.async_copy(xh.at[g], rws, sem)

        def fire(nb):
            @pl.when(nb > 0)
            def _():
                issue(0, gsrc0, gslot0, rows0, sem0)

            def pair(k, _):
                b0 = 2 * k
                b1 = b0 + 1
                b2 = b0 + 2

                @pl.when(b1 < nb)
                def _():
                    issue(b1, gsrc1, gslot1, rows1, sem1)

                @pl.when(b0 < nb)
                def _():
                    pltpu.make_async_copy(xh.at[gsrc0], rows0, sem0).wait()
                    pltpu.sync_copy(rows0, acc_out.at[gslot0], add=True)

                @pl.when(b2 < nb)
                def _():
                    issue(b2, gsrc0, gslot0, rows0, sem0)

                @pl.when(b1 < nb)
                def _():
                    pltpu.make_async_copy(xh.at[gsrc1], rows1, sem1).wait()
                    pltpu.sync_copy(rows1, acc_out.at[gslot1], add=True)
                return 0
            lax.fori_loop(0, (nb + 1) // 2, pair, 0)

        def group(g, f):
            def chunk(ci, f):
                off = (g * GRP + ci) * ECH
                pltpu.sync_copy(esrc.at[pl.ds(off, ECH)], srcb)
                pltpu.sync_copy(slots.at[pl.ds(off, ECH)], slotb)

                def v(i, f):
                    s16 = srcb[pl.ds(i * 16, 16)]
                    t16 = slotb[pl.ds(i * 16, 16)]
                    d16 = t16 & 0xFFFF
                    r16 = lax.shift_right_logical(t16, 16)
                    m = (d16 >= dlo) & (d16 < dlo + TPW)
                    loc = r16 * NPADR + d16
                    return _compact16(stage_s, stage_t, s16, loc, m, f)
                return lax.fori_loop(0, VSTEPS, v, f)
            fv = lax.fori_loop(0, GRP, chunk, f)
            fs = fv[0]
            nb = fs // BATCH
            fire(nb)
            rem = fs - nb * BATCH
            for j in range(BATCH // 16):
                @pl.when(16 * j < rem)
                def _():
                    stage_s[pl.ds(16 * j, 16)] = (
                        stage_s[pl.ds(nb * BATCH + 16 * j, 16)])
                    stage_t[pl.ds(16 * j, 16)] = (
                        stage_t[pl.ds(nb * BATCH + 16 * j, 16)])
            return jnp.full((16,), rem, jnp.int32)
        fv = lax.fori_loop(0, NGRP, group, jnp.zeros((16,), jnp.int32))
        f = fv[0]

        nb = (f + BATCH - 1) // BATCH
        pe = nb * BATCH
        dums = jnp.zeros((16,), jnp.int32)
        dumt = jnp.full((16,), trash, jnp.int32)
        for j in range(BATCH // 16):
            @pl.when(f + 16 * j < pe)
            def _():
                stage_s[pl.ds(f + 16 * j, 16)] = dums
                stage_t[pl.ds(f + 16 * j, 16)] = dumt
        fire(nb)

    return agg


NB = 400
NBLK = N // NB


def _layer_body(x_ref, acc_ref, cnt_ref, w_ref, b_ref):
    inv = 1.0 / jnp.maximum(cnt_ref[...], 1.0)
    o = jnp.dot(x_ref[...], w_ref[0], preferred_element_type=jnp.float32)
    for r in range(R):
        o = o + jnp.dot(acc_ref[r] * inv[r][:, ONES_COL:ONES_COL + 1],
                        w_ref[r + 1], preferred_element_type=jnp.float32)
    return jnp.maximum(o + b_ref[...], 0.0)


@functools.lru_cache(maxsize=None)
def _make_layer(din, dout, final):
    def body(x_ref, acc_ref, cnt_ref, w_ref, b_ref, o_ref):
        h = _layer_body(x_ref, acc_ref, cnt_ref, w_ref, b_ref)
        if not final:
            o_ref[...] = h
        else:
            i = pl.program_id(0)
            t = jnp.max(jnp.tanh(h), axis=0, keepdims=True)

            @pl.when(i == 0)
            def _():
                o_ref[...] = jnp.full((1, dout), -2.0, jnp.float32)
            o_ref[...] = jnp.maximum(o_ref[...], t)

            @pl.when(i == NBLK - 1)
            def _():
                z = o_ref[...]
                o_ref[...] = z * lax.rsqrt(jnp.sum(z * z))

    out_shape = jax.ShapeDtypeStruct((1, dout) if final else (N, dout),
                                     jnp.float32)
    out_spec = (pl.BlockSpec((1, dout), lambda i: (0, 0)) if final
                else pl.BlockSpec((NB, dout), lambda i: (i, 0)))
    return pl.pallas_call(
        body,
        grid=(NBLK,),
        in_specs=[
            pl.BlockSpec((NB, din), lambda i: (i, 0)),
            pl.BlockSpec((R, NB, din), lambda i: (0, i, 0)),
            pl.BlockSpec((R, NB, 128), lambda i: (0, i, 0)),
            pl.BlockSpec((R + 1, din, dout), lambda i: (0, 0, 0)),
            pl.BlockSpec((1, dout), lambda i: (0, 0)),
        ],
        out_specs=out_spec,
        out_shape=out_shape,
        compiler_params=pltpu.CompilerParams(
            dimension_semantics=("arbitrary",)),
    )


def _stack_w(root, w, din):
    pad = din - root.shape[0]
    rootp = jnp.pad(root, ((0, pad), (0, 0)))
    wp = jnp.pad(w, ((0, 0), (0, pad), (0, 0)))
    return jnp.concatenate([rootp[None], wp], axis=0)


def kernel(x, edge_index, edge_type, W1, root1, b1, W2, root2, b2,
           W3, root3, b3):
    x256 = jnp.pad(x, ((0, 0), (0, 256 - x.shape[1])))
    x256 = x256.at[:, ONES_COL].set(1.0)
    npadE = EPAD - E
    esrc = jnp.concatenate([edge_index[0], jnp.zeros((npadE,), jnp.int32)])
    edst = jnp.concatenate(
        [edge_index[1], jnp.full((npadE,), DUMMY_DST, jnp.int32)])
    et = jnp.concatenate([edge_type, jnp.zeros((npadE,), jnp.int32)])

    slots = _make_prep()(edst, et)

    acc1 = _make_agg(256)(x256, esrc, slots)[:R * NPADR].reshape(
        R, NPADR, 256)
    cnt = acc1
    h1 = _make_layer(256, 256, False)(
        x256, acc1, cnt, _stack_w(root1, W1, 256), b1[None])

    acc2 = _make_agg(256)(h1, esrc, slots)[:R * NPADR].reshape(
        R, NPADR, 256)
    h2 = _make_layer(256, 256, False)(
        h1, acc2, cnt, _stack_w(root2, W2, 256), b2[None])

    acc3 = _make_agg(256)(h2, esrc, slots)[:R * NPADR].reshape(
        R, NPADR, 256)
    z = _make_layer(256, 512, True)(
        h2, acc3, cnt, _stack_w(root3, W3, 256), b3[None])
    return z


# speedup vs baseline: 14.6314x; 1.3479x over previous
import functools

import jax
import jax.numpy as jnp
from jax import lax
from jax.experimental import pallas as pl
from jax.experimental.pallas import tpu as pltpu
from jax.experimental.pallas import tpu_sc as plsc

N = 50000
E = 800000
R = 6
NC = 2
NS = 16
NW = NC * NS
NPADR = 50176
TPW = NPADR // NW
EPW = NPADR
EPAD = EPW * NS
ECH = 1568
NCH = EPW // ECH
NCHG = EPAD // ECH
VSTEPS = ECH // 16
BATCH = 128
GRP = 16
NGRP = NCHG // GRP
SCAP = GRP * ECH + 256
STRASH = SCAP - 16
DUMMY_DST = 50000
ONES_COL = 32
RT = R * NPADR + NW * 8

_MESH = dict(core_axis_name="c", subcore_axis_name="s", num_cores=NC,
             num_subcores=NS)


def _zero16():
    return jnp.zeros((16,), jnp.float32)


def _compact16(stage_s, stage_t, s16, loc, m, f):
    pos = plsc.cumsum(m.astype(jnp.int32)) - 1
    lane = lax.iota(jnp.int32, 16)
    idx = jnp.where(m, f + pos, STRASH + lane)
    if stage_s is not None:
        plsc.store_scatter(stage_s, [idx], s16)
    plsc.store_scatter(stage_t, [idx], loc)
    return f + plsc.all_reduce_population_count(m)


CAP = EPAD + BATCH


@functools.lru_cache(maxsize=None)
def _make_prep():
    mesh = plsc.VectorSubcoreMesh(**_MESH)

    @functools.partial(
        pl.kernel,
        out_type=(
            jax.ShapeDtypeStruct((NW * CAP,), jnp.int32),
            jax.ShapeDtypeStruct((NW * CAP,), jnp.int32),
            jax.ShapeDtypeStruct((NW * 16,), jnp.int32),
        ),
        mesh=mesh,
        compiler_params=pltpu.CompilerParams(needs_layout_passes=False),
        scratch_types=[
            pltpu.VMEM((ECH,), jnp.int32),
            pltpu.VMEM((ECH,), jnp.int32),
            pltpu.VMEM((ECH,), jnp.int32),
            pltpu.VMEM((SCAP,), jnp.int32),
            pltpu.VMEM((SCAP,), jnp.int32),
            pltpu.VMEM((16,), jnp.int32),
        ],
    )
    def prep(esrc, edst, et, bsrc, bslot, ncnt,
             srcb, dstb, typb, stage_s, stage_t, nbuf):
        sc = lax.axis_index("c")
        sid = lax.axis_index("s")
        w = sid * NC + sc
        dlo = w * TPW
        trash = R * NPADR + w * 8

        def flush(nb, tb):
            def wr(b, _):
                pltpu.sync_copy(
                    stage_s.at[pl.ds(b * BATCH, BATCH)],
                    bsrc.at[pl.ds(w * CAP + (tb + b) * BATCH, BATCH)])
                pltpu.sync_copy(
                    stage_t.at[pl.ds(b * BATCH, BATCH)],
                    bslot.at[pl.ds(w * CAP + (tb + b) * BATCH, BATCH)])
                return 0
            lax.fori_loop(0, nb, wr, 0)

        def group(g, carry):
            f, tb = carry

            def chunk(ci, f):
                off = (g * GRP + ci) * ECH
                pltpu.sync_copy(esrc.at[pl.ds(off, ECH)], srcb)
                pltpu.sync_copy(edst.at[pl.ds(off, ECH)], dstb)
                pltpu.sync_copy(et.at[pl.ds(off, ECH)], typb)

                def v(i, f):
                    s16 = srcb[pl.ds(i * 16, 16)]
                    d16 = dstb[pl.ds(i * 16, 16)]
                    r16 = typb[pl.ds(i * 16, 16)]
                    m = (d16 >= dlo) & (d16 < dlo + TPW)
                    loc = r16 * NPADR + d16
                    return _compact16(stage_s, stage_t, s16, loc, m, f)
                return lax.fori_loop(0, VSTEPS, v, f)
            fv = lax.fori_loop(0, GRP, chunk, f)
            fs = fv[0]
            nb = fs // BATCH
            tbs = tb[0]
            flush(nb, tbs)
            rem = fs - nb * BATCH
            for j in range(BATCH // 16):
                @pl.when(16 * j < rem)
                def _():
                    stage_s[pl.ds(16 * j, 16)] = (
                        stage_s[pl.ds(nb * BATCH + 16 * j, 16)])
                    stage_t[pl.ds(16 * j, 16)] = (
                        stage_t[pl.ds(nb * BATCH + 16 * j, 16)])
            return (jnp.full((16,), rem, jnp.int32), tb + nb)
        f, tb = lax.fori_loop(
            0, NGRP, group,
            (jnp.zeros((16,), jnp.int32), jnp.zeros((16,), jnp.int32)))
        fs = f[0]
        tbs = tb[0]

        nb = (fs + BATCH - 1) // BATCH
        pe = nb * BATCH
        dums = jnp.zeros((16,), jnp.int32)
        dumt = jnp.full((16,), trash, jnp.int32)
        for j in range(BATCH // 16):
            @pl.when(fs + 16 * j < pe)
            def _():
                stage_s[pl.ds(fs + 16 * j, 16)] = dums
                stage_t[pl.ds(fs + 16 * j, 16)] = dumt
        flush(nb, tbs)
        nbuf[pl.ds(0, 16)] = tb + nb
        pltpu.sync_copy(nbuf, ncnt.at[pl.ds(w * 16, 16)])

    return prep


@functools.lru_cache(maxsize=None)
def _make_agg(din):
    zr = 16
    nz = TPW // zr
    mesh = plsc.VectorSubcoreMesh(**_MESH)

    @functools.partial(
        pl.kernel,
        out_type=jax.ShapeDtypeStruct((RT, din), jnp.float32),
        mesh=mesh,
        compiler_params=pltpu.CompilerParams(needs_layout_passes=False),
        scratch_types=[
            pltpu.VMEM((BATCH,), jnp.int32),
            pltpu.VMEM((BATCH,), jnp.int32),
            pltpu.VMEM((BATCH,), jnp.int32),
            pltpu.VMEM((BATCH,), jnp.int32),
            pltpu.VMEM((BATCH, din), jnp.float32),
            pltpu.VMEM((BATCH, din), jnp.float32),
            pltpu.VMEM((zr, din), jnp.float32),
            pltpu.VMEM((16,), jnp.int32),
            pltpu.SemaphoreType.DMA,
            pltpu.SemaphoreType.DMA,
        ],
    )
    def agg(xh, bsrc, bslot, ncnt, acc_out,
            gsrc0, gslot0, gsrc1, gslot1, rows0, rows1, zbuf, nbuf,
            sem0, sem1):
        sc = lax.axis_index("c")
        sid = lax.axis_index("s")
        w = sid * NC + sc
        dlo = w * TPW

        def fill_z(j, _):
            for q in range(din // 16):
                zbuf[j, pl.ds(q * 16, 16)] = _zero16()
            return 0
        lax.fori_loop(0, zr, fill_z, 0)

        for r in range(R):
            base = r * NPADR + dlo
            for zi in range(nz):
                pltpu.sync_copy(zbuf, acc_out.at[pl.ds(base + zi * zr, zr)])
        trash = R * NPADR + w * 8
        pltpu.sync_copy(zbuf.at[pl.ds(0, 8)], acc_out.at[pl.ds(trash, 8)])

        pltpu.sync_copy(ncnt.at[pl.ds(w * 16, 16)], nbuf)
        nb = nbuf[pl.ds(0, 16)][0]

        def issue(b, g, t, rws, sem):
            pltpu.sync_copy(bsrc.at[pl.ds(w * CAP + b * BATCH, BATCH)], g)
            pltpu.sync_copy(bslot.at[pl.ds(w * CAP + b * BATCH, BATCH)], t)
            pltpu.async_copy(xh.at[g], rws, sem)

        @pl.when(nb > 0)
        def _():
            issue(0, gsrc0, gslot0, rows0, sem0)

        def pair(k, _):
            b0 = 2 * k
            b1 = b0 + 1
            b2 = b0 + 2

            @pl.when(b1 < nb)
            def _():
                issue(b1, gsrc1, gslot1, rows1, sem1)

            @pl.when(b0 < nb)
            def _():
                pltpu.make_async_copy(xh.at[gsrc0], rows0, sem0).wait()
                pltpu.sync_copy(rows0, acc_out.at[gslot0], add=True)

            @pl.when(b2 < nb)
            def _():
                issue(b2, gsrc0, gslot0, rows0, sem0)

            @pl.when(b1 < nb)
            def _():
                pltpu.make_async_copy(xh.at[gsrc1], rows1, sem1).wait()
                pltpu.sync_copy(rows1, acc_out.at[gslot1], add=True)
            return 0
        lax.fori_loop(0, (nb + 1) // 2, pair, 0)

    return agg


NB = 400
NBLK = N // NB


def _layer_body(x_ref, acc_ref, cnt_ref, w_ref, b_ref):
    cnt = jnp.maximum(cnt_ref[...], 1.0)
    o = jnp.dot(x_ref[...], w_ref[0], preferred_element_type=jnp.float32)
    for r in range(R):
        o = o + jnp.dot(acc_ref[r] / cnt[r][:, ONES_COL:ONES_COL + 1],
                        w_ref[r + 1], preferred_element_type=jnp.float32)
    return jnp.maximum(o + b_ref[...], 0.0)


@functools.lru_cache(maxsize=None)
def _make_layer(din, dout, final):
    def body(x_ref, acc_ref, cnt_ref, w_ref, b_ref, o_ref):
        h = _layer_body(x_ref, acc_ref, cnt_ref, w_ref, b_ref)
        if not final:
            o_ref[...] = h
        else:
            i = pl.program_id(0)
            t = jnp.max(jnp.tanh(h), axis=0, keepdims=True)

            @pl.when(i == 0)
            def _():
                o_ref[...] = jnp.full((1, dout), -2.0, jnp.float32)
            o_ref[...] = jnp.maximum(o_ref[...], t)

            @pl.when(i == NBLK - 1)
            def _():
                z = o_ref[...]
                o_ref[...] = z / jnp.sqrt(jnp.sum(z * z))

    out_shape = jax.ShapeDtypeStruct((1, dout) if final else (N, dout),
                                     jnp.float32)
    out_spec = (pl.BlockSpec((1, dout), lambda i: (0, 0)) if final
                else pl.BlockSpec((NB, dout), lambda i: (i, 0)))
    return pl.pallas_call(
        body,
        grid=(NBLK,),
        in_specs=[
            pl.BlockSpec((NB, din), lambda i: (i, 0)),
            pl.BlockSpec((R, NB, din), lambda i: (0, i, 0)),
            pl.BlockSpec((R, NB, 128), lambda i: (0, i, 0)),
            pl.BlockSpec((R + 1, din, dout), lambda i: (0, 0, 0)),
            pl.BlockSpec((1, dout), lambda i: (0, 0)),
        ],
        out_specs=out_spec,
        out_shape=out_shape,
        compiler_params=pltpu.CompilerParams(
            dimension_semantics=("arbitrary",)),
    )


def _stack_w(root, w, din):
    pad = din - root.shape[0]
    rootp = jnp.pad(root, ((0, pad), (0, 0)))
    wp = jnp.pad(w, ((0, 0), (0, pad), (0, 0)))
    return jnp.concatenate([rootp[None], wp], axis=0)


def kernel(x, edge_index, edge_type, W1, root1, b1, W2, root2, b2,
           W3, root3, b3):
    x256 = jnp.pad(x, ((0, 0), (0, 256 - x.shape[1])))
    x256 = x256.at[:, ONES_COL].set(1.0)
    npadE = EPAD - E
    esrc = jnp.concatenate([edge_index[0], jnp.zeros((npadE,), jnp.int32)])
    edst = jnp.concatenate(
        [edge_index[1], jnp.full((npadE,), DUMMY_DST, jnp.int32)])
    et = jnp.concatenate([edge_type, jnp.zeros((npadE,), jnp.int32)])

    bsrc, bslot, ncnt = _make_prep()(esrc, edst, et)

    acc1 = _make_agg(256)(x256, bsrc, bslot, ncnt)[:R * NPADR].reshape(
        R, NPADR, 256)
    cnt = acc1
    h1 = _make_layer(256, 256, False)(
        x256, acc1, cnt, _stack_w(root1, W1, 256), b1[None])

    acc2 = _make_agg(256)(h1, bsrc, bslot, ncnt)[:R * NPADR].reshape(
        R, NPADR, 256)
    h2 = _make_layer(256, 256, False)(
        h1, acc2, cnt, _stack_w(root2, W2, 256), b2[None])

    acc3 = _make_agg(256)(h2, bsrc, bslot, ncnt)[:R * NPADR].reshape(
        R, NPADR, 256)
    z = _make_layer(256, 512, True)(
        h2, acc3, cnt, _stack_w(root3, W3, 256), b3[None])
    return z


# speedup vs baseline: 14.8863x; 1.0174x over previous
import functools

import jax
import jax.numpy as jnp
from jax import lax
from jax.experimental import pallas as pl
from jax.experimental.pallas import tpu as pltpu
from jax.experimental.pallas import tpu_sc as plsc

N = 50000
E = 800000
R = 6
NC = 2
NS = 16
NW = NC * NS
NPADR = 50176
TPW = NPADR // NW
EPW = NPADR
EPAD = EPW * NS
ECH = 1568
NCH = EPW // ECH
NCHG = EPAD // ECH
VSTEPS = ECH // 16
BATCH = 128
GRP = 16
NGRP = NCHG // GRP
SCAP = GRP * ECH + 256
STRASH = SCAP - 16
DUMMY_DST = 50000
ONES_COL = 32
RT = R * NPADR + NW * 8

_MESH = dict(core_axis_name="c", subcore_axis_name="s", num_cores=NC,
             num_subcores=NS)


def _zero16():
    return jnp.zeros((16,), jnp.float32)


def _compact16(stage_s, stage_t, s16, loc, m, f):
    pos = plsc.cumsum(m.astype(jnp.int32)) - 1
    lane = lax.iota(jnp.int32, 16)
    idx = jnp.where(m, f + pos, STRASH + lane)
    if stage_s is not None:
        plsc.store_scatter(stage_s, [idx], s16)
    plsc.store_scatter(stage_t, [idx], loc)
    return f + plsc.all_reduce_population_count(m)


CAP = EPAD + BATCH


@functools.lru_cache(maxsize=None)
def _make_prep():
    mesh = plsc.VectorSubcoreMesh(**_MESH)

    @functools.partial(
        pl.kernel,
        out_type=(
            jax.ShapeDtypeStruct((NW * CAP,), jnp.int32),
            jax.ShapeDtypeStruct((NW * CAP,), jnp.int32),
            jax.ShapeDtypeStruct((NW * 16,), jnp.int32),
        ),
        mesh=mesh,
        compiler_params=pltpu.CompilerParams(needs_layout_passes=False),
        scratch_types=[
            pltpu.VMEM((ECH,), jnp.int32),
            pltpu.VMEM((ECH,), jnp.int32),
            pltpu.VMEM((ECH,), jnp.int32),
            pltpu.VMEM((SCAP,), jnp.int32),
            pltpu.VMEM((SCAP,), jnp.int32),
            pltpu.VMEM((16,), jnp.int32),
        ],
    )
    def prep(esrc, edst, et, bsrc, bslot, ncnt,
             srcb, dstb, typb, stage_s, stage_t, nbuf):
        sc = lax.axis_index("c")
        sid = lax.axis_index("s")
        w = sid * NC + sc
        dlo = w * TPW
        trash = R * NPADR + w * 8

        def flush(nb, tb):
            def wr(b, _):
                pltpu.sync_copy(
                    stage_s.at[pl.ds(b * BATCH, BATCH)],
                    bsrc.at[pl.ds(w * CAP + (tb + b) * BATCH, BATCH)])
                pltpu.sync_copy(
                    stage_t.at[pl.ds(b * BATCH, BATCH)],
                    bslot.at[pl.ds(w * CAP + (tb + b) * BATCH, BATCH)])
                return 0
            lax.fori_loop(0, nb, wr, 0)

        def group(g, carry):
            f, tb = carry

            def chunk(ci, f):
                off = (g * GRP + ci) * ECH
                pltpu.sync_copy(esrc.at[pl.ds(off, ECH)], srcb)
                pltpu.sync_copy(edst.at[pl.ds(off, ECH)], dstb)
                pltpu.sync_copy(et.at[pl.ds(off, ECH)], typb)

                def v(i, f):
                    s16 = srcb[pl.ds(i * 16, 16)]
                    d16 = dstb[pl.ds(i * 16, 16)]
                    r16 = typb[pl.ds(i * 16, 16)]
                    m = (d16 >= dlo) & (d16 < dlo + TPW)
                    loc = r16 * NPADR + d16
                    return _compact16(stage_s, stage_t, s16, loc, m, f)
                return lax.fori_loop(0, VSTEPS, v, f)
            fv = lax.fori_loop(0, GRP, chunk, f)
            fs = fv[0]
            nb = fs // BATCH
            tbs = tb[0]
            flush(nb, tbs)
            rem = fs - nb * BATCH
            for j in range(BATCH // 16):
                @pl.when(16 * j < rem)
                def _():
                    stage_s[pl.ds(16 * j, 16)] = (
                        stage_s[pl.ds(nb * BATCH + 16 * j, 16)])
                    stage_t[pl.ds(16 * j, 16)] = (
                        stage_t[pl.ds(nb * BATCH + 16 * j, 16)])
            return (jnp.full((16,), rem, jnp.int32), tb + nb)
        f, tb = lax.fori_loop(
            0, NGRP, group,
            (jnp.zeros((16,), jnp.int32), jnp.zeros((16,), jnp.int32)))
        fs = f[0]
        tbs = tb[0]

        nb = (fs + BATCH - 1) // BATCH
        pe = nb * BATCH
        dums = jnp.zeros((16,), jnp.int32)
        dumt = jnp.full((16,), trash, jnp.int32)
        for j in range(BATCH // 16):
            @pl.when(fs + 16 * j < pe)
            def _():
                stage_s[pl.ds(fs + 16 * j, 16)] = dums
                stage_t[pl.ds(fs + 16 * j, 16)] = dumt
        flush(nb, tbs)
        nbuf[pl.ds(0, 16)] = tb + nb
        pltpu.sync_copy(nbuf, ncnt.at[pl.ds(w * 16, 16)])

    return prep


@functools.lru_cache(maxsize=None)
def _make_agg(din):
    zr = 112
    nz = TPW // zr
    mesh = plsc.VectorSubcoreMesh(**_MESH)

    @functools.partial(
        pl.kernel,
        out_type=jax.ShapeDtypeStruct((RT, din), jnp.float32),
        mesh=mesh,
        compiler_params=pltpu.CompilerParams(needs_layout_passes=False),
        scratch_types=[
            pltpu.VMEM((BATCH,), jnp.int32),
            pltpu.VMEM((BATCH,), jnp.int32),
            pltpu.VMEM((BATCH,), jnp.int32),
            pltpu.VMEM((BATCH,), jnp.int32),
            pltpu.VMEM((BATCH, din), jnp.float32),
            pltpu.VMEM((BATCH, din), jnp.float32),
            pltpu.VMEM((zr, din), jnp.float32),
            pltpu.VMEM((16,), jnp.int32),
            pltpu.SemaphoreType.DMA,
            pltpu.SemaphoreType.DMA,
        ],
    )
    def agg(xh, bsrc, bslot, ncnt, acc_out,
            gsrc0, gslot0, gsrc1, gslot1, rows0, rows1, zbuf, nbuf,
            sem0, sem1):
        sc = lax.axis_index("c")
        sid = lax.axis_index("s")
        w = sid * NC + sc
        dlo = w * TPW

        def fill_z(j, _):
            for q in range(din // 16):
                zbuf[j, pl.ds(q * 16, 16)] = _zero16()
            return 0
        lax.fori_loop(0, zr, fill_z, 0)

        for r in range(R):
            base = r * NPADR + dlo
            for zi in range(nz):
                pltpu.sync_copy(zbuf, acc_out.at[pl.ds(base + zi * zr, zr)])
        trash = R * NPADR + w * 8
        pltpu.sync_copy(zbuf.at[pl.ds(0, 8)], acc_out.at[pl.ds(trash, 8)])

        pltpu.sync_copy(ncnt.at[pl.ds(w * 16, 16)], nbuf)
        nb = nbuf[pl.ds(0, 16)][0]

        def issue(b, g, t, rws, sem):
            pltpu.sync_copy(bsrc.at[pl.ds(w * CAP + b * BATCH, BATCH)], g)
            pltpu.sync_copy(bslot.at[pl.ds(w * CAP + b * BATCH, BATCH)], t)
            pltpu.async_copy(xh.at[g], rws, sem)

        @pl.when(nb > 0)
        def _():
            issue(0, gsrc0, gslot0, rows0, sem0)

        def pair(k, _):
            b0 = 2 * k
            b1 = b0 + 1
            b2 = b0 + 2

            @pl.when(b1 < nb)
            def _():
                issue(b1, gsrc1, gslot1, rows1, sem1)

            @pl.when(b0 < nb)
            def _():
                pltpu.make_async_copy(xh.at[gsrc0], rows0, sem0).wait()
                pltpu.sync_copy(rows0, acc_out.at[gslot0], add=True)

            @pl.when(b2 < nb)
            def _():
                issue(b2, gsrc0, gslot0, rows0, sem0)

            @pl.when(b1 < nb)
            def _():
                pltpu.make_async_copy(xh.at[gsrc1], rows1, sem1).wait()
                pltpu.sync_copy(rows1, acc_out.at[gslot1], add=True)
            return 0
        lax.fori_loop(0, (nb + 1) // 2, pair, 0)

    return agg


NB = 400
NBLK = N // NB


def _layer_body(x_ref, acc_ref, cnt_ref, w_ref, b_ref):
    cnt = jnp.maximum(cnt_ref[...], 1.0)
    o = jnp.dot(x_ref[...], w_ref[0], preferred_element_type=jnp.float32)
    for r in range(R):
        o = o + jnp.dot(acc_ref[r] / cnt[r][:, ONES_COL:ONES_COL + 1],
                        w_ref[r + 1], preferred_element_type=jnp.float32)
    return jnp.maximum(o + b_ref[...], 0.0)


@functools.lru_cache(maxsize=None)
def _make_layer(din, dout, final):
    def body(x_ref, acc_ref, cnt_ref, w_ref, b_ref, o_ref):
        h = _layer_body(x_ref, acc_ref, cnt_ref, w_ref, b_ref)
        if not final:
            o_ref[...] = h
        else:
            i = pl.program_id(0)
            t = jnp.max(jnp.tanh(h), axis=0, keepdims=True)

            @pl.when(i == 0)
            def _():
                o_ref[...] = jnp.full((1, dout), -2.0, jnp.float32)
            o_ref[...] = jnp.maximum(o_ref[...], t)

            @pl.when(i == NBLK - 1)
            def _():
                z = o_ref[...]
                o_ref[...] = z / jnp.sqrt(jnp.sum(z * z))

    out_shape = jax.ShapeDtypeStruct((1, dout) if final else (N, dout),
                                     jnp.float32)
    out_spec = (pl.BlockSpec((1, dout), lambda i: (0, 0)) if final
                else pl.BlockSpec((NB, dout), lambda i: (i, 0)))
    return pl.pallas_call(
        body,
        grid=(NBLK,),
        in_specs=[
            pl.BlockSpec((NB, din), lambda i: (i, 0)),
            pl.BlockSpec((R, NB, din), lambda i: (0, i, 0)),
            pl.BlockSpec((R, NB, 128), lambda i: (0, i, 0)),
            pl.BlockSpec((R + 1, din, dout), lambda i: (0, 0, 0)),
            pl.BlockSpec((1, dout), lambda i: (0, 0)),
        ],
        out_specs=out_spec,
        out_shape=out_shape,
        compiler_params=pltpu.CompilerParams(
            dimension_semantics=("arbitrary",)),
    )


def _stack_w(root, w, din):
    pad = din - root.shape[0]
    rootp = jnp.pad(root, ((0, pad), (0, 0)))
    wp = jnp.pad(w, ((0, 0), (0, pad), (0, 0)))
    return jnp.concatenate([rootp[None], wp], axis=0)


def kernel(x, edge_index, edge_type, W1, root1, b1, W2, root2, b2,
           W3, root3, b3):
    x256 = jnp.pad(x, ((0, 0), (0, 256 - x.shape[1])))
    x256 = x256.at[:, ONES_COL].set(1.0)
    npadE = EPAD - E
    esrc = jnp.concatenate([edge_index[0], jnp.zeros((npadE,), jnp.int32)])
    edst = jnp.concatenate(
        [edge_index[1], jnp.full((npadE,), DUMMY_DST, jnp.int32)])
    et = jnp.concatenate([edge_type, jnp.zeros((npadE,), jnp.int32)])

    bsrc, bslot, ncnt = _make_prep()(esrc, edst, et)

    acc1 = _make_agg(256)(x256, bsrc, bslot, ncnt)[:R * NPADR].reshape(
        R, NPADR, 256)
    cnt = acc1
    h1 = _make_layer(256, 256, False)(
        x256, acc1, cnt, _stack_w(root1, W1, 256), b1[None])

    acc2 = _make_agg(256)(h1, bsrc, bslot, ncnt)[:R * NPADR].reshape(
        R, NPADR, 256)
    h2 = _make_layer(256, 256, False)(
        h1, acc2, cnt, _stack_w(root2, W2, 256), b2[None])

    acc3 = _make_agg(256)(h2, bsrc, bslot, ncnt)[:R * NPADR].reshape(
        R, NPADR, 256)
    z = _make_layer(256, 512, True)(
        h2, acc3, cnt, _stack_w(root3, W3, 256), b3[None])
    return z
